# baseline (speedup 1.0000x reference)
import functools

import jax
import jax.numpy as jnp
from jax import lax
from jax.experimental import pallas as pl
from jax.experimental.pallas import tpu as pltpu
from jax.experimental.pallas import tpu_sc as plsc

NC = 2
NS = 16
NW = NC * NS
LANES = 8

F32 = jnp.float32
BF16 = jnp.bfloat16


def _pack_bf16_pairs(v):
    k = v.shape[-1] // 2
    u = jax.lax.bitcast_convert_type(v, jnp.uint32)
    r = u + jnp.uint32(0x7FFF) + ((u >> 16) & jnp.uint32(1))
    lo = r[:, :k] >> 16
    hi = r[:, k:] & jnp.uint32(0xFFFF0000)
    return jax.lax.bitcast_convert_type(lo | hi, jnp.int32)


def _unpack_bf16_pairs(p):
    u = jax.lax.bitcast_convert_type(p, jnp.uint32)
    lo = jax.lax.bitcast_convert_type(u << 16, F32)
    hi = jax.lax.bitcast_convert_type(u & jnp.uint32(0xFFFF0000), F32)
    return jnp.concatenate([lo, hi], axis=-1)


def _pq_body(x_ref, wa_ref, wb_ref, p_ref, q_ref):
    x = x_ref[...]
    p_ref[...] = _pack_bf16_pairs(
        jnp.dot(x, wa_ref[...], preferred_element_type=F32))
    q_ref[...] = _pack_bf16_pairs(
        jnp.dot(x, wb_ref[...], preferred_element_type=F32))


def _make_pq(n_nodes, d_feat, filters, blk):
    grid = n_nodes // blk
    return pl.pallas_call(
        _pq_body,
        grid=(grid,),
        in_specs=[
            pl.BlockSpec((blk, d_feat), lambda i: (i, 0)),
            pl.BlockSpec((d_feat, filters), lambda i: (0, 0)),
            pl.BlockSpec((d_feat, filters), lambda i: (0, 0)),
        ],
        out_specs=[
            pl.BlockSpec((blk, filters // 2), lambda i: (i, 0)),
            pl.BlockSpec((blk, filters // 2), lambda i: (i, 0)),
        ],
        out_shape=[
            jax.ShapeDtypeStruct((n_nodes, filters // 2), jnp.int32),
            jax.ShapeDtypeStruct((n_nodes, filters // 2), jnp.int32),
        ],
    )


def _gather_body(chunk, n_chunks, p_hbm, q_hbm, src_hbm, dst_hbm,
                 s1_hbm, s2_hbm, isv, idv,
                 av0, bv0, av1, bv1, av2, bv2,
                 semg0, semg1, semg2, semo0, semo1, semo2):
    wid = lax.axis_index("s") * NC + lax.axis_index("c")
    per_w = n_chunks // NW
    edge0 = wid * per_w * chunk

    pltpu.sync_copy(src_hbm.at[pl.ds(edge0, per_w * chunk)], isv)
    pltpu.sync_copy(dst_hbm.at[pl.ds(edge0, per_w * chunk)], idv)

    sets = ((av0, bv0, semg0, semo0), (av1, bv1, semg1, semo1),
            (av2, bv2, semg2, semo2))

    def fire(k, s):
        av, bv, semg, _ = s
        ix = isv.at[pl.ds(k * chunk, chunk)]
        iy = idv.at[pl.ds(k * chunk, chunk)]
        pltpu.async_copy(p_hbm.at[ix], av, semg)
        pltpu.async_copy(q_hbm.at[iy], bv, semg)

    def wait_gather(k, s):
        av, bv, semg, _ = s
        ix = isv.at[pl.ds(k * chunk, chunk)]
        iy = idv.at[pl.ds(k * chunk, chunk)]
        pltpu.make_async_copy(p_hbm.at[ix], av, semg).wait()
        pltpu.make_async_copy(q_hbm.at[iy], bv, semg).wait()

    def fire_out(k, s):
        av, bv, _, semo = s
        pltpu.async_copy(av, s1_hbm.at[pl.ds(edge0 + k * chunk, chunk)],
                         semo)
        pltpu.async_copy(bv, s2_hbm.at[pl.ds(edge0 + k * chunk, chunk)],
                         semo)

    def wait_out(k, s):
        av, bv, _, semo = s
        pltpu.make_async_copy(
            av, s1_hbm.at[pl.ds(edge0 + k * chunk, chunk)], semo).wait()
        pltpu.make_async_copy(
            bv, s2_hbm.at[pl.ds(edge0 + k * chunk, chunk)], semo).wait()

    def step(k, s, last):
        wait_gather(k, s)
        fire_out(k, s)
        if not last:
            def refill():
                wait_out(k, s)
                fire(k + 3, s)
            pl.when(k + 3 < per_w)(refill)

    fire(0, sets[0])
    fire(1, sets[1])
    fire(2, sets[2])

    def triple(g, _):
        step(3 * g, sets[0], False)
        step(3 * g + 1, sets[1], False)
        step(3 * g + 2, sets[2], False)
        return 0

    lax.fori_loop(0, per_w // 3, triple, 0)
    for k in range((per_w // 3) * 3, per_w):
        step(jnp.int32(k), sets[k % 3], True)
    for k in range(per_w - 3, per_w):
        wait_out(jnp.int32(k), sets[k % 3])


def _make_gather(n_nodes, filters, n_edges, chunk):
    n_chunks = n_edges // chunk
    per_w = n_chunks // NW
    mesh = plsc.VectorSubcoreMesh(
        core_axis_name="c", subcore_axis_name="s",
        num_cores=NC, num_subcores=NS)
    buf = lambda: pltpu.VMEM((chunk, filters // 2), jnp.int32)
    return pl.kernel(
        functools.partial(_gather_body, chunk, n_chunks),
        compiler_params=pltpu.CompilerParams(use_tc_tiling_on_sc=False),
        out_type=[
            jax.ShapeDtypeStruct((n_edges, filters // 2), jnp.int32),
            jax.ShapeDtypeStruct((n_edges, filters // 2), jnp.int32),
        ],
        mesh=mesh,
        scratch_types=[
            pltpu.VMEM((per_w * chunk,), jnp.int32),
            pltpu.VMEM((per_w * chunk,), jnp.int32),
            buf(), buf(), buf(), buf(), buf(), buf(),
            pltpu.SemaphoreType.DMA,
            pltpu.SemaphoreType.DMA,
            pltpu.SemaphoreType.DMA,
            pltpu.SemaphoreType.DMA,
            pltpu.SemaphoreType.DMA,
            pltpu.SemaphoreType.DMA,
        ],
    )


def _leaky(h):
    return jnp.where(h > 0, h, 0.01 * h)


def _mlp_body(s1_ref, s2_ref, e_ref, w1c_ref, b1_ref, w2_ref, b2_ref,
              w3_ref, b3_ref, w4_ref, b4_ref, m_ref):
    h = (_unpack_bf16_pairs(s1_ref[...]) + _unpack_bf16_pairs(s2_ref[...])
         + jnp.dot(e_ref[...], w1c_ref[...], preferred_element_type=F32)
         + b1_ref[...])
    h = _leaky(h).astype(BF16)
    h = _leaky(jnp.dot(h, w2_ref[...], preferred_element_type=F32)
               + b2_ref[...]).astype(BF16)
    h = _leaky(jnp.dot(h, w3_ref[...], preferred_element_type=F32)
               + b3_ref[...]).astype(BF16)
    m_ref[...] = jnp.dot(h, w4_ref[...],
                         preferred_element_type=F32) + b4_ref[...]


def _make_mlp(n_edges, d_edge, filters, out_dim, blk):
    grid = n_edges // blk
    full = lambda r, c: pl.BlockSpec((r, c), lambda i: (0, 0))
    return pl.pallas_call(
        _mlp_body,
        grid=(grid,),
        in_specs=[
            pl.BlockSpec((blk, filters // 2), lambda i: (i, 0)),
            pl.BlockSpec((blk, filters // 2), lambda i: (i, 0)),
            pl.BlockSpec((blk, d_edge), lambda i: (i, 0)),
            full(d_edge, filters),
            full(1, filters),
            full(filters, filters),
            full(1, filters),
            full(filters, filters),
            full(1, filters),
            full(filters, out_dim),
            full(1, out_dim),
        ],
        out_specs=pl.BlockSpec((blk, out_dim), lambda i: (i, 0)),
        out_shape=jax.ShapeDtypeStruct((n_edges, out_dim), F32),
    )


def _scatter_body(chunk, n_nodes, out_dim, n_chunks, m_hbm, dst2d_hbm,
                  part_hbm, idv, mv0, mv1, zv, acc_shared, sem0, sem1):
    cid = lax.axis_index("c")
    sid = lax.axis_index("s")
    wid = sid * NC + cid
    per_w = n_chunks // NW
    edge0 = wid * per_w * chunk
    n_blocks = n_nodes // 80

    def zero_row(i, _):
        for j in range(out_dim // 16):
            zv[i, pl.ds(j * 16, 16)] = jnp.zeros((16,), F32)
        return 0

    lax.fori_loop(0, 80, zero_row, 0)

    for j in range((n_blocks + NS - 1) // NS):
        b = sid + j * NS

        @pl.when(b < n_blocks)
        def _():
            pltpu.sync_copy(zv, acc_shared.at[pl.ds(b * 80, 80)])

    plsc.subcore_barrier()

    pltpu.sync_copy(dst2d_hbm.at[wid], idv)

    sets = ((mv0, sem0), (mv1, sem1))

    def fire(k, s):
        mv, sem = s
        pltpu.async_copy(m_hbm.at[pl.ds(edge0 + k * chunk, chunk)], mv, sem)

    def step(k, s, last):
        mv, sem = s
        pltpu.make_async_copy(
            m_hbm.at[pl.ds(edge0 + k * chunk, chunk)], mv, sem).wait()
        pltpu.sync_copy(mv, acc_shared.at[idv.at[k]], add=True)
        if not last:
            pl.when(k + 2 < per_w)(lambda: fire(k + 2, s))

    fire(0, sets[0])
    fire(1, sets[1])

    def pair(g, _):
        step(2 * g, sets[0], False)
        step(2 * g + 1, sets[1], False)
        return 0

    lax.fori_loop(0, per_w // 2, pair, 0)
    step(jnp.int32(per_w - 1), sets[0], True)
    plsc.subcore_barrier()
    for j in range((n_blocks + NS - 1) // NS):
        b = sid + j * NS

        @pl.when(b < n_blocks)
        def _():
            pltpu.sync_copy(acc_shared.at[pl.ds(b * 80, 80)],
                            part_hbm.at[cid, pl.ds(b * 80, 80)])


def _make_scatter(n_nodes, out_dim, n_edges, chunk):
    n_chunks = n_edges // chunk
    per_w = n_chunks // NW
    mesh = plsc.VectorSubcoreMesh(
        core_axis_name="c", subcore_axis_name="s",
        num_cores=NC, num_subcores=NS)
    return pl.kernel(
        functools.partial(_scatter_body, chunk, n_nodes, out_dim, n_chunks),
        out_type=jax.ShapeDtypeStruct((NC, n_nodes, out_dim), F32),
        mesh=mesh,
        scratch_types=[
            pltpu.VMEM((per_w, chunk), jnp.int32),
            pltpu.VMEM((chunk, out_dim), F32),
            pltpu.VMEM((chunk, out_dim), F32),
            pltpu.VMEM((80, out_dim), F32),
            pltpu.VMEM_SHARED((n_nodes, out_dim), F32),
            pltpu.SemaphoreType.DMA,
            pltpu.SemaphoreType.DMA,
        ],
    )


def _combine_body(p_ref, o_ref):
    o_ref[...] = p_ref[0] + p_ref[1]


def _make_combine(n_nodes, out_dim, blk):
    return pl.pallas_call(
        _combine_body,
        grid=(n_nodes // blk,),
        in_specs=[pl.BlockSpec((NC, blk, out_dim), lambda i: (0, i, 0))],
        out_specs=pl.BlockSpec((blk, out_dim), lambda i: (i, 0)),
        out_shape=jax.ShapeDtypeStruct((n_nodes, out_dim), F32),
    )


def kernel(x, edge_index, edge_attr, W1, b1, W2, b2, W3, b3, W4, b4):
    n_nodes, d_feat = x.shape
    n_edges = edge_index.shape[1]
    d_edge = edge_attr.shape[1]
    filters = W2.shape[0]
    out_dim = W4.shape[1]

    idx = edge_index.astype(jnp.int32)
    src = idx[0]
    dst = idx[1]
    w1a = W1[:d_feat]
    w1b = W1[d_feat:2 * d_feat]
    w1c = W1[2 * d_feat:]

    p, q = _make_pq(n_nodes, d_feat, filters, blk=2000)(x, w1a, w1b)
    s1, s2 = _make_gather(n_nodes, filters, n_edges, chunk=80)(p, q, src, dst)
    m = _make_mlp(n_edges, d_edge, filters, out_dim, blk=2000)(
        s1, s2, edge_attr.astype(BF16), w1c.astype(BF16), b1.reshape(1, -1),
        W2.astype(BF16), b2.reshape(1, -1),
        W3.astype(BF16), b3.reshape(1, -1),
        W4.astype(BF16), b4.reshape(1, -1))
    dst2d = dst.reshape(NW, n_edges // (NW * 80), 80)
    part = _make_scatter(n_nodes, out_dim, n_edges, chunk=80)(m, dst2d)
    out = _make_combine(n_nodes, out_dim, blk=2000)(part)
    return out


# speedup vs baseline: 1.2281x; 1.2281x over previous
import functools

import jax
import jax.numpy as jnp
import numpy as np
from jax import lax
from jax.experimental import pallas as pl
from jax.experimental.pallas import tpu as pltpu
from jax.experimental.pallas import tpu_sc as plsc

NC = 2
NS = 16
NW = NC * NS
LANES = 8

F32 = jnp.float32
BF16 = jnp.bfloat16


def _pack_bf16_pairs(v):
    k = v.shape[-1] // 2
    u = jax.lax.bitcast_convert_type(v, jnp.uint32)
    r = u + jnp.uint32(0x7FFF) + ((u >> 16) & jnp.uint32(1))
    lo = r[:, :k] >> 16
    hi = r[:, k:] & jnp.uint32(0xFFFF0000)
    return jax.lax.bitcast_convert_type(lo | hi, jnp.int32)


def _unpack_bf16_pairs(p):
    u = jax.lax.bitcast_convert_type(p, jnp.uint32)
    lo = jax.lax.bitcast_convert_type(u << 16, F32)
    hi = jax.lax.bitcast_convert_type(u & jnp.uint32(0xFFFF0000), F32)
    return jnp.concatenate([lo, hi], axis=-1)


def _pq_body(x_ref, wa_ref, wb_ref, p_ref, q_ref):
    x = x_ref[...]
    p_ref[...] = _pack_bf16_pairs(
        jnp.dot(x, wa_ref[...], preferred_element_type=F32))
    q_ref[...] = _pack_bf16_pairs(
        jnp.dot(x, wb_ref[...], preferred_element_type=F32))


def _make_pq(n_nodes, d_feat, filters, blk):
    grid = n_nodes // blk
    return pl.pallas_call(
        _pq_body,
        grid=(grid,),
        in_specs=[
            pl.BlockSpec((blk, d_feat), lambda i: (i, 0)),
            pl.BlockSpec((d_feat, filters), lambda i: (0, 0)),
            pl.BlockSpec((d_feat, filters), lambda i: (0, 0)),
        ],
        out_specs=[
            pl.BlockSpec((blk, filters // 2), lambda i: (i, 0)),
            pl.BlockSpec((blk, filters // 2), lambda i: (i, 0)),
        ],
        out_shape=[
            jax.ShapeDtypeStruct((n_nodes, filters // 2), jnp.int32),
            jax.ShapeDtypeStruct((n_nodes, filters // 2), jnp.int32),
        ],
    )


def _gather_body(chunk, n_chunks, n_nodes, filters, p_hbm, q_hbm, src_hbm,
                 dst_hbm, s1_hbm, s2_hbm, isv, idv,
                 av0, bv0, av1, bv1, av2, bv2,
                 semg0, semg1, semg2, semo0, semo1, semo2):
    wid = lax.axis_index("s") * NC + lax.axis_index("c")
    per_w = n_chunks // NW
    edge0 = wid * per_w * chunk

    pltpu.sync_copy(src_hbm.at[pl.ds(edge0, per_w * chunk)], isv)
    pltpu.sync_copy(dst_hbm.at[pl.ds(edge0, per_w * chunk)], idv)

    sets = ((av0, bv0, semg0, semo0), (av1, bv1, semg1, semo1),
            (av2, bv2, semg2, semo2))

    def fire(k, s):
        av, bv, semg, _ = s
        ix = isv.at[pl.ds(k * chunk, chunk)]
        iy = idv.at[pl.ds(k * chunk, chunk)]
        pltpu.async_copy(p_hbm.at[ix], av, semg)
        pltpu.async_copy(q_hbm.at[iy], bv, semg)

    def wait_gather(k, s):
        av, bv, semg, _ = s
        ix = isv.at[pl.ds(k * chunk, chunk)]
        iy = idv.at[pl.ds(k * chunk, chunk)]
        pltpu.make_async_copy(p_hbm.at[ix], av, semg).wait()
        pltpu.make_async_copy(q_hbm.at[iy], bv, semg).wait()

    def fire_out(k, s):
        av, bv, _, semo = s
        sl = pl.ds(edge0 + k * chunk, chunk)
        pltpu.async_copy(av, s1_hbm.at[sl], semo)
        pltpu.async_copy(bv, s2_hbm.at[sl], semo)

    def wait_out(k, s):
        av, bv, _, semo = s
        sl = pl.ds(edge0 + k * chunk, chunk)
        pltpu.make_async_copy(av, s1_hbm.at[sl], semo).wait()
        pltpu.make_async_copy(bv, s2_hbm.at[sl], semo).wait()

    def step(k, s, last):
        wait_gather(k, s)
        fire_out(k, s)
        if not last:
            def refill():
                wait_out(k, s)
                fire(k + 3, s)
            pl.when(k + 3 < per_w)(refill)

    fire(0, sets[0])
    fire(1, sets[1])
    fire(2, sets[2])

    def triple(g, _):
        step(3 * g, sets[0], False)
        step(3 * g + 1, sets[1], False)
        step(3 * g + 2, sets[2], False)
        return 0

    lax.fori_loop(0, per_w // 3, triple, 0)
    for k in range((per_w // 3) * 3, per_w):
        step(jnp.int32(k), sets[k % 3], True)
    for k in range(per_w - 3, per_w):
        wait_out(jnp.int32(k), sets[k % 3])


def _make_gather(n_nodes, filters, n_edges, chunk):
    n_chunks = n_edges // chunk
    per_w = n_chunks // NW
    mesh = plsc.VectorSubcoreMesh(
        core_axis_name="c", subcore_axis_name="s",
        num_cores=NC, num_subcores=NS)
    buf = lambda: pltpu.VMEM((chunk, filters // 2), jnp.int32)
    return pl.kernel(
        functools.partial(_gather_body, chunk, n_chunks, n_nodes, filters),
        compiler_params=pltpu.CompilerParams(use_tc_tiling_on_sc=False),
        out_type=[
            jax.ShapeDtypeStruct((n_edges, filters // 2), jnp.int32),
            jax.ShapeDtypeStruct((n_edges, filters // 2), jnp.int32),
        ],
        mesh=mesh,
        scratch_types=[
            pltpu.VMEM((per_w * chunk,), jnp.int32),
            pltpu.VMEM((per_w * chunk,), jnp.int32),
            buf(), buf(), buf(), buf(), buf(), buf(),
            pltpu.SemaphoreType.DMA,
            pltpu.SemaphoreType.DMA,
            pltpu.SemaphoreType.DMA,
            pltpu.SemaphoreType.DMA,
            pltpu.SemaphoreType.DMA,
            pltpu.SemaphoreType.DMA,
        ],
    )


def _leaky(h):
    return jnp.where(h > 0, h, 0.01 * h)


def _split_even_odd(p):
    c2 = p.shape[-1] // 2
    u = jax.lax.bitcast_convert_type(p, jnp.uint32)
    lo = jax.lax.bitcast_convert_type(u << 16, F32)
    hi = jax.lax.bitcast_convert_type(u & jnp.uint32(0xFFFF0000), F32)
    even = jnp.concatenate([lo[:, :c2], hi[:, :c2]], axis=-1)
    odd = jnp.concatenate([lo[:, c2:], hi[:, c2:]], axis=-1)
    return even, odd


def _mlp_body(s1_ref, s2_ref, e_ref, w1c_ref, b1_ref, w2_ref, b2_ref,
              w3_ref, b3_ref, w4_ref, b4_ref, m_ref):
    p_e, p_o = _split_even_odd(s1_ref[...])
    q_e, q_o = _split_even_odd(s2_ref[...])
    a_e, a_o = _split_even_odd(e_ref[...])
    s = jnp.concatenate([p_e + q_e, p_o + q_o], axis=0)
    ea = jnp.concatenate([a_e, a_o], axis=0).astype(BF16)
    h = (s + jnp.dot(ea, w1c_ref[...], preferred_element_type=F32)
         + b1_ref[...])
    h = _leaky(h).astype(BF16)
    h = _leaky(jnp.dot(h, w2_ref[...], preferred_element_type=F32)
               + b2_ref[...]).astype(BF16)
    h = _leaky(jnp.dot(h, w3_ref[...], preferred_element_type=F32)
               + b3_ref[...]).astype(BF16)
    m_ref[...] = jnp.dot(h, w4_ref[...],
                         preferred_element_type=F32) + b4_ref[...]


def _make_mlp(n_edges, d_edge, filters, out_dim, blk):
    grid = n_edges // blk
    full = lambda r, c: pl.BlockSpec((r, c), lambda i: (0, 0))
    return pl.pallas_call(
        _mlp_body,
        grid=(grid,),
        in_specs=[
            pl.BlockSpec((blk // 2, filters), lambda i: (i, 0)),
            pl.BlockSpec((blk // 2, filters), lambda i: (i, 0)),
            pl.BlockSpec((blk // 2, d_edge), lambda i: (i, 0)),
            full(d_edge, filters),
            full(1, filters),
            full(filters, filters),
            full(1, filters),
            full(filters, filters),
            full(1, filters),
            full(filters, out_dim),
            full(1, out_dim),
        ],
        out_specs=pl.BlockSpec((blk, out_dim), lambda i: (i, 0)),
        out_shape=jax.ShapeDtypeStruct((n_edges, out_dim), F32),
    )


def _scatter_body(chunk, n_nodes, out_dim, n_chunks, m_hbm, dst2d_hbm,
                  part_hbm, idv, mv0, mv1, zv, acc_shared, sem0, sem1):
    cid = lax.axis_index("c")
    sid = lax.axis_index("s")
    wid = sid * NC + cid
    per_w = n_chunks // NW
    edge0 = wid * per_w * chunk
    n_blocks = n_nodes // 80

    def zero_row(i, _):
        for j in range(out_dim // 16):
            zv[i, pl.ds(j * 16, 16)] = jnp.zeros((16,), F32)
        return 0

    lax.fori_loop(0, 80, zero_row, 0)

    for j in range((n_blocks + NS - 1) // NS):
        b = sid + j * NS

        @pl.when(b < n_blocks)
        def _():
            pltpu.sync_copy(zv, acc_shared.at[pl.ds(b * 80, 80)])

    plsc.subcore_barrier()

    pltpu.sync_copy(dst2d_hbm.at[wid], idv)

    sets = ((mv0, sem0), (mv1, sem1))

    def fire(k, s):
        mv, sem = s
        pltpu.async_copy(m_hbm.at[pl.ds(edge0 + k * chunk, chunk)], mv, sem)

    def step(k, s, last):
        mv, sem = s
        pltpu.make_async_copy(
            m_hbm.at[pl.ds(edge0 + k * chunk, chunk)], mv, sem).wait()
        pltpu.sync_copy(mv, acc_shared.at[idv.at[k]], add=True)
        if not last:
            pl.when(k + 2 < per_w)(lambda: fire(k + 2, s))

    fire(0, sets[0])
    fire(1, sets[1])

    def pair(g, _):
        step(2 * g, sets[0], False)
        step(2 * g + 1, sets[1], False)
        return 0

    lax.fori_loop(0, per_w // 2, pair, 0)
    step(jnp.int32(per_w - 1), sets[0], True)
    plsc.subcore_barrier()
    for j in range((n_blocks + NS - 1) // NS):
        b = sid + j * NS

        @pl.when(b < n_blocks)
        def _():
            pltpu.sync_copy(acc_shared.at[pl.ds(b * 80, 80)],
                            part_hbm.at[cid, pl.ds(b * 80, 80)])


def _make_scatter(n_nodes, out_dim, n_edges, chunk):
    n_chunks = n_edges // chunk
    per_w = n_chunks // NW
    mesh = plsc.VectorSubcoreMesh(
        core_axis_name="c", subcore_axis_name="s",
        num_cores=NC, num_subcores=NS)
    return pl.kernel(
        functools.partial(_scatter_body, chunk, n_nodes, out_dim, n_chunks),
        out_type=jax.ShapeDtypeStruct((NC, n_nodes, out_dim), F32),
        mesh=mesh,
        scratch_types=[
            pltpu.VMEM((per_w, chunk), jnp.int32),
            pltpu.VMEM((chunk, out_dim), F32),
            pltpu.VMEM((chunk, out_dim), F32),
            pltpu.VMEM((80, out_dim), F32),
            pltpu.VMEM_SHARED((n_nodes, out_dim), F32),
            pltpu.SemaphoreType.DMA,
            pltpu.SemaphoreType.DMA,
        ],
    )


def _combine_body(p_ref, o_ref):
    o_ref[...] = p_ref[0] + p_ref[1]


def _make_combine(n_nodes, out_dim, blk):
    return pl.pallas_call(
        _combine_body,
        grid=(n_nodes // blk,),
        in_specs=[pl.BlockSpec((NC, blk, out_dim), lambda i: (0, i, 0))],
        out_specs=pl.BlockSpec((blk, out_dim), lambda i: (i, 0)),
        out_shape=jax.ShapeDtypeStruct((n_nodes, out_dim), F32),
    )


def kernel(x, edge_index, edge_attr, W1, b1, W2, b2, W3, b3, W4, b4):
    n_nodes, d_feat = x.shape
    n_edges = edge_index.shape[1]
    d_edge = edge_attr.shape[1]
    filters = W2.shape[0]
    out_dim = W4.shape[1]

    idx = edge_index.astype(jnp.int32)
    src = idx[0]
    dst = idx[1]
    w1a = W1[:d_feat]
    w1b = W1[d_feat:2 * d_feat]
    w1c = W1[2 * d_feat:]

    blk = 2000
    p, q = _make_pq(n_nodes, d_feat, filters, blk=blk)(x, w1a, w1b)
    s1, s2 = _make_gather(n_nodes, filters, n_edges, chunk=80)(
        p, q, src, dst)
    s1 = s1.reshape(n_edges // 2, filters)
    s2 = s2.reshape(n_edges // 2, filters)
    ea2 = _pack_bf16_pairs(edge_attr).reshape(n_edges // 2, d_edge)
    m = _make_mlp(n_edges, d_edge, filters, out_dim, blk=blk)(
        s1, s2, ea2, w1c.astype(BF16), b1.reshape(1, -1),
        W2.astype(BF16), b2.reshape(1, -1),
        W3.astype(BF16), b3.reshape(1, -1),
        W4.astype(BF16), b4.reshape(1, -1))
    base = np.arange(0, n_edges, blk, dtype=np.int32)
    within = np.concatenate([np.arange(0, blk, 2, dtype=np.int32),
                             np.arange(1, blk, 2, dtype=np.int32)])
    perm = (base[:, None] + within[None, :]).reshape(-1)
    dst_perm = dst[perm]
    dst2d = dst_perm.reshape(NW, n_edges // (NW * 80), 80)
    part = _make_scatter(n_nodes, out_dim, n_edges, chunk=80)(m, dst2d)
    out = _make_combine(n_nodes, out_dim, blk=2000)(part)
    return out


# speedup vs baseline: 1.5880x; 1.2931x over previous
import functools

import jax
import jax.numpy as jnp
import numpy as np
from jax import lax
from jax.experimental import pallas as pl
from jax.experimental.pallas import tpu as pltpu
from jax.experimental.pallas import tpu_sc as plsc

NC = 2
NS = 16
NW = NC * NS
LANES = 8

F32 = jnp.float32
BF16 = jnp.bfloat16


def _pack_bf16_pairs(v):
    k = v.shape[-1] // 2
    u = jax.lax.bitcast_convert_type(v, jnp.uint32)
    r = u + jnp.uint32(0x7FFF) + ((u >> 16) & jnp.uint32(1))
    lo = r[:, :k] >> 16
    hi = r[:, k:] & jnp.uint32(0xFFFF0000)
    return jax.lax.bitcast_convert_type(lo | hi, jnp.int32)


def _unpack_bf16_pairs(p):
    u = jax.lax.bitcast_convert_type(p, jnp.uint32)
    lo = jax.lax.bitcast_convert_type(u << 16, F32)
    hi = jax.lax.bitcast_convert_type(u & jnp.uint32(0xFFFF0000), F32)
    return jnp.concatenate([lo, hi], axis=-1)


def _pq_body(x_ref, wa_ref, wb_ref, p_ref, q_ref):
    x = x_ref[...]
    p_ref[...] = jnp.dot(x, wa_ref[...], preferred_element_type=F32)
    q_ref[...] = jnp.dot(x, wb_ref[...], preferred_element_type=F32)


def _make_pq(n_nodes, d_feat, filters, blk):
    grid = n_nodes // blk
    return pl.pallas_call(
        _pq_body,
        grid=(grid,),
        in_specs=[
            pl.BlockSpec((blk, d_feat), lambda i: (i, 0)),
            pl.BlockSpec((d_feat, filters), lambda i: (0, 0)),
            pl.BlockSpec((d_feat, filters), lambda i: (0, 0)),
        ],
        out_specs=[
            pl.BlockSpec((blk, filters), lambda i: (i, 0)),
            pl.BlockSpec((blk, filters), lambda i: (i, 0)),
        ],
        out_shape=[
            jax.ShapeDtypeStruct((n_nodes, filters), F32),
            jax.ShapeDtypeStruct((n_nodes, filters), F32),
        ],
    )


def _gather_body(chunk, n_chunks, p_hbm, q_hbm, src_hbm, dst_hbm,
                 s_hbm, isv, idv, av0, bv0, ov0, av1, bv1, ov1,
                 semg0, semg1, semo0, semo1):
    wid = lax.axis_index("s") * NC + lax.axis_index("c")
    per_w = n_chunks // NW
    edge0 = wid * per_w * chunk

    pltpu.sync_copy(src_hbm.at[pl.ds(edge0, per_w * chunk)], isv)
    pltpu.sync_copy(dst_hbm.at[pl.ds(edge0, per_w * chunk)], idv)

    sets = ((av0, bv0, ov0, semg0, semo0), (av1, bv1, ov1, semg1, semo1))

    def fire(k, s):
        av, bv, _, semg, _ = s
        ix = isv.at[pl.ds(k * chunk, chunk)]
        iy = idv.at[pl.ds(k * chunk, chunk)]
        pltpu.async_copy(p_hbm.at[ix], av, semg)
        pltpu.async_copy(q_hbm.at[iy], bv, semg)

    def wait_gather(k, s):
        av, bv, _, semg, _ = s
        ix = isv.at[pl.ds(k * chunk, chunk)]
        iy = idv.at[pl.ds(k * chunk, chunk)]
        pltpu.make_async_copy(p_hbm.at[ix], av, semg).wait()
        pltpu.make_async_copy(q_hbm.at[iy], bv, semg).wait()

    def wait_out(k, s):
        _, _, ov, _, semo = s
        pltpu.make_async_copy(
            ov, s_hbm.at[pl.ds(edge0 + k * chunk, chunk)], semo).wait()

    def step(k, s, last):
        av, bv, ov, _, semo = s
        wait_gather(k, s)
        pl.when(k >= 2)(lambda: wait_out(k - 2, s))

        def add_row(i, _):
            for j in range(LANES):
                sl = pl.ds(j * 16, 16)
                ov[i, sl] = av[i, sl] + bv[i, sl]
            return 0

        lax.fori_loop(0, chunk, add_row, 0)
        pltpu.async_copy(ov, s_hbm.at[pl.ds(edge0 + k * chunk, chunk)], semo)
        if not last:
            pl.when(k + 2 < per_w)(lambda: fire(k + 2, s))

    fire(0, sets[0])
    fire(1, sets[1])

    def pair(g, _):
        step(2 * g, sets[0], False)
        step(2 * g + 1, sets[1], False)
        return 0

    lax.fori_loop(0, per_w // 2, pair, 0)
    step(jnp.int32(per_w - 1), sets[0], True)
    wait_out(jnp.int32(per_w - 1), sets[0])
    wait_out(jnp.int32(per_w - 2), sets[1])


def _make_gather(n_nodes, filters, n_edges, chunk):
    n_chunks = n_edges // chunk
    per_w = n_chunks // NW
    mesh = plsc.VectorSubcoreMesh(
        core_axis_name="c", subcore_axis_name="s",
        num_cores=NC, num_subcores=NS)
    buf = lambda: pltpu.VMEM((chunk, filters), F32)
    return pl.kernel(
        functools.partial(_gather_body, chunk, n_chunks),
        out_type=jax.ShapeDtypeStruct((n_edges, filters), F32),
        mesh=mesh,
        scratch_types=[
            pltpu.VMEM((per_w * chunk,), jnp.int32),
            pltpu.VMEM((per_w * chunk,), jnp.int32),
            buf(), buf(), buf(), buf(), buf(), buf(),
            pltpu.SemaphoreType.DMA,
            pltpu.SemaphoreType.DMA,
            pltpu.SemaphoreType.DMA,
            pltpu.SemaphoreType.DMA,
        ],
    )


def _leaky(h):
    return jnp.where(h > 0, h, 0.01 * h)


def _mlp_body(s_ref, e_ref, w1c_ref, b1_ref, w2_ref, b2_ref,
              w3_ref, b3_ref, w4_ref, b4_ref, m_ref):
    h = (s_ref[...]
         + jnp.dot(e_ref[...], w1c_ref[...], preferred_element_type=F32)
         + b1_ref[...])
def _make_mlp(n_edges, d_edge, filters, out_dim, blk):
    grid = n_edges // blk
    full = lambda r, c: pl.BlockSpec((r, c), lambda i: (0, 0))
    return pl.pallas_call(
        _mlp_body,
        grid=(grid,),
        in_specs=[
            pl.BlockSpec((blk, filters), lambda i: (i, 0)),
            pl.BlockSpec((blk, d_edge), lambda i: (i, 0)),
            full(d_edge, filters),
            full(1, filters),
            full(filters, filters),
            full(1, filters),
            full(filters, filters),
            full(1, filters),
            full(filters, out_dim),
            full(1, out_dim),
        ],
        out_specs=pl.BlockSpec((blk, out_dim), lambda i: (i, 0)),
        out_shape=jax.ShapeDtypeStruct((n_edges, out_dim), F32),
    )


def _scatter_body(chunk, n_nodes, out_dim, n_chunks, m_hbm, dst2d_hbm,
                  part_hbm, idv, mv0, mv1, zv, acc_shared, sem0, sem1):
    cid = lax.axis_index("c")
    sid = lax.axis_index("s")
    wid = sid * NC + cid
    per_w = n_chunks // NW
    edge0 = wid * per_w * chunk
    n_blocks = n_nodes // 80

    def zero_row(i, _):
        for j in range(out_dim // 16):
            zv[i, pl.ds(j * 16, 16)] = jnp.zeros((16,), F32)
        return 0

    lax.fori_loop(0, 80, zero_row, 0)

    for j in range((n_blocks + NS - 1) // NS):
        b = sid + j * NS

        @pl.when(b < n_blocks)
        def _():
            pltpu.sync_copy(zv, acc_shared.at[pl.ds(b * 80, 80)])

    plsc.subcore_barrier()

    pltpu.sync_copy(dst2d_hbm.at[wid], idv)

    sets = ((mv0, sem0), (mv1, sem1))

    def fire(k, s):
        mv, sem = s
        pltpu.async_copy(m_hbm.at[pl.ds(edge0 + k * chunk, chunk)], mv, sem)

    def step(k, s, last):
        mv, sem = s
        pltpu.make_async_copy(
            m_hbm.at[pl.ds(edge0 + k * chunk, chunk)], mv, sem).wait()
        pltpu.sync_copy(mv, acc_shared.at[idv.at[k]], add=True)
        if not last:
            pl.when(k + 2 < per_w)(lambda: fire(k + 2, s))

    fire(0, sets[0])
    fire(1, sets[1])

    def pair(g, _):
        step(2 * g, sets[0], False)
        step(2 * g + 1, sets[1], False)
        return 0

    lax.fori_loop(0, per_w // 2, pair, 0)
    step(jnp.int32(per_w - 1), sets[0], True)
    plsc.subcore_barrier()
    for j in range((n_blocks + NS - 1) // NS):
        b = sid + j * NS

        @pl.when(b < n_blocks)
        def _():
            pltpu.sync_copy(acc_shared.at[pl.ds(b * 80, 80)],
                            part_hbm.at[cid, pl.ds(b * 80, 80)])


def _make_scatter(n_nodes, out_dim, n_edges, chunk):
    n_chunks = n_edges // chunk
    per_w = n_chunks // NW
    mesh = plsc.VectorSubcoreMesh(
        core_axis_name="c", subcore_axis_name="s",
        num_cores=NC, num_subcores=NS)
    return pl.kernel(
        functools.partial(_scatter_body, chunk, n_nodes, out_dim, n_chunks),
        out_type=jax.ShapeDtypeStruct((NC, n_nodes, out_dim), F32),
        mesh=mesh,
        scratch_types=[
            pltpu.VMEM((per_w, chunk), jnp.int32),
            pltpu.VMEM((chunk, out_dim), F32),
            pltpu.VMEM((chunk, out_dim), F32),
            pltpu.VMEM((80, out_dim), F32),
            pltpu.VMEM_SHARED((n_nodes, out_dim), F32),
            pltpu.SemaphoreType.DMA,
            pltpu.SemaphoreType.DMA,
        ],
    )


def _combine_body(p_ref, o_ref):
    o_ref[...] = p_ref[0] + p_ref[1]


def _make_combine(n_nodes, out_dim, blk):
    return pl.pallas_call(
        _combine_body,
        grid=(n_nodes // blk,),
        in_specs=[pl.BlockSpec((NC, blk, out_dim), lambda i: (0, i, 0))],
        out_specs=pl.BlockSpec((blk, out_dim), lambda i: (i, 0)),
        out_shape=jax.ShapeDtypeStruct((n_nodes, out_dim), F32),
    )


def kernel(x, edge_index, edge_attr, W1, b1, W2, b2, W3, b3, W4, b4):
    n_nodes, d_feat = x.shape
    n_edges = edge_index.shape[1]
    d_edge = edge_attr.shape[1]
    filters = W2.shape[0]
    out_dim = W4.shape[1]

    idx = edge_index.astype(jnp.int32)
    src = idx[0]
    dst = idx[1]
    w1a = W1[:d_feat]
    w1b = W1[d_feat:2 * d_feat]
    w1c = W1[2 * d_feat:]

    blk = 2000
    p, q = _make_pq(n_nodes, d_feat, filters, blk=blk)(x, w1a, w1b)
    s = _make_gather(n_nodes, filters, n_edges, chunk=80)(p, q, src, dst)
    m = _make_mlp(n_edges, d_edge, filters, out_dim, blk=blk)(
        s, edge_attr.astype(BF16), w1c.astype(BF16), b1.reshape(1, -1),
        W2.astype(BF16), b2.reshape(1, -1),
        W3.astype(BF16), b3.reshape(1, -1),
        W4.astype(BF16), b4.reshape(1, -1))
    dst2d = dst.reshape(NW, n_edges // (NW * 80), 80)
    part = _make_scatter(n_nodes, out_dim, n_edges, chunk=80)(m, dst2d)
    out = _make_combine(n_nodes, out_dim, blk=2000)(part)
    return out
